# baseline (speedup 1.0000x reference)
import functools

import jax
import jax.numpy as jnp
from jax import lax
from jax.experimental import pallas as pl
from jax.experimental.pallas import tpu as pltpu
from jax.experimental.pallas import tpu_sc as plsc

B = 16384
S = 25
F = 128
E = 128
L = 16
FW = F // 2
NC, NS = 2, 16
NW = NC * NS
BPW = B // NW
CH = 8
NCHUNK = BPW // CH
NBUF = 2
SG = B // (NW * 128)

def _lo_f32(x):
    return plsc.bitcast(x << 16, jnp.float32)


def _hi_f32(x):
    return plsc.bitcast(x & -65536, jnp.float32)


def _accum_chunk(nrows, obuf):
    for d in range(CH):
        r0 = d * S
        for j in range(FW // L):
            sl = pl.ds(j * L, L)
            acc = plsc.bitcast(nrows[r0, sl], jnp.bfloat16)
            for s in range(1, S):
                acc = acc + plsc.bitcast(nrows[r0 + s, sl], jnp.bfloat16)
            obuf[d, sl] = plsc.bitcast(acc, jnp.int32)


@functools.cache
def _build_sc_gather():
  mesh = plsc.VectorSubcoreMesh(core_axis_name="c", subcore_axis_name="s")

  @functools.partial(
    pl.kernel,
    out_type=jax.ShapeDtypeStruct((B, F), jnp.int32),
    mesh=mesh,
    compiler_params=pltpu.CompilerParams(
        needs_layout_passes=False, use_tc_tiling_on_sc=False),
    scratch_types=[
        pltpu.VMEM((SG, 128), jnp.int32),
        pltpu.VMEM((NCHUNK, CH * S), jnp.int32),
        pltpu.VMEM((2, 128, FW), jnp.int32),
        [pltpu.VMEM((CH * S, FW), jnp.int32)] * NBUF,
        [pltpu.VMEM((CH, FW), jnp.int32)] * NBUF,
        [pltpu.SemaphoreType.DMA] * NBUF,
        [pltpu.SemaphoreType.DMA] * NBUF,
        [pltpu.SemaphoreType.DMA] * 2,
        [pltpu.SemaphoreType.DMA] * 2,
    ],
)
  def _sc_gather(nodes2, neigh2, feat, comb_out,
                 nidx, eidx, srows, nrows, obufs, gsems, wsems, sgsems, swsems):
      wid = lax.axis_index("s") * NC + lax.axis_index("c")
      obase = wid * BPW

      pltpu.sync_copy(nodes2.at[pl.ds(wid * SG, SG)], nidx)
      pltpu.sync_copy(neigh2.at[pl.ds(wid * NCHUNK, NCHUNK)], eidx)

      for c in range(NBUF):
          pltpu.make_async_copy(feat.at[eidx.at[c]], nrows[c], gsems[c]).start()

      pltpu.make_async_copy(feat.at[nidx.at[0]], srows.at[0], sgsems[0]).start()
      pltpu.make_async_copy(feat.at[nidx.at[1]], srows.at[1], sgsems[1]).start()
      for g in range(SG):
          p = g % 2
          pltpu.make_async_copy(feat.at[nidx.at[g]], srows.at[p], sgsems[p]).wait()
          out_sl = comb_out.at[pl.ds(obase + g * 128, 128), pl.ds(0, FW)]
          pltpu.make_async_copy(srows.at[p], out_sl, swsems[p]).start()
          if g + 2 < SG:
              pltpu.make_async_copy(srows.at[p], out_sl, swsems[p]).wait()
              pltpu.make_async_copy(feat.at[nidx.at[g + 2]], srows.at[p], sgsems[p]).start()
      for g in range(SG - 2, SG):
          p = g % 2
          out_sl = comb_out.at[pl.ds(obase + g * 128, 128), pl.ds(0, FW)]
          pltpu.make_async_copy(srows.at[p], out_sl, swsems[p]).wait()

      def body(c2, carry):
          for k in range(NBUF):
              c = c2 * NBUF + k

              @pl.when(c >= NBUF)
              def _wait_write():
                  dst = comb_out.at[pl.ds(obase + (c - NBUF) * CH, CH),
                                    pl.ds(FW, FW)]
                  pltpu.make_async_copy(obufs[k], dst, wsems[k]).wait()

              pltpu.make_async_copy(feat.at[eidx.at[c]], nrows[k], gsems[k]).wait()
              _accum_chunk(nrows[k], obufs[k])

              @pl.when(c + NBUF < NCHUNK)
              def _next_gather():
                  pltpu.make_async_copy(
                      feat.at[eidx.at[c + NBUF]], nrows[k], gsems[k]).start()

              dst = comb_out.at[pl.ds(obase + c * CH, CH), pl.ds(FW, FW)]
              pltpu.make_async_copy(obufs[k], dst, wsems[k]).start()
          return carry

      lax.fori_loop(0, NCHUNK // NBUF, body, 0)

      for c in range(NCHUNK - NBUF, NCHUNK):
          k = c % NBUF
          dst = comb_out.at[pl.ds(obase + c * CH, CH), pl.ds(FW, FW)]
          pltpu.make_async_copy(obufs[k], dst, wsems[k]).wait()

  return _sc_gather


def _pack_body(x_ref, o_ref):
    u = lax.bitcast_convert_type(x_ref[...], jnp.int32)
    def rnd(v):
        lsb = lax.shift_right_logical(v, 16) & 1
        return lax.shift_right_logical(v + 0x7FFF + lsb, 16)
    w = rnd(u[:, :FW]) | (rnd(u[:, FW:]) << 16)
    o_ref[:, :FW] = w[:_PBLK // 2]
    o_ref[:, FW:] = w[_PBLK // 2:]


_PBLK = 10000
N_NODES_ = 100000


@jax.jit
def _tc_pack(features):
    packed = pl.pallas_call(
        _pack_body,
        out_shape=jax.ShapeDtypeStruct((N_NODES_ // 2, F), jnp.int32),
        grid=(N_NODES_ // _PBLK,),
        in_specs=[pl.BlockSpec((_PBLK, F), lambda i: (i, 0))],
        out_specs=pl.BlockSpec((_PBLK // 2, F), lambda i: (i, 0)),
    )(features)
    return packed.reshape(N_NODES_, FW)


def _remap_idx(n):
    r = n % _PBLK
    return (n - r) + jnp.where(r < _PBLK // 2, 2 * r, 2 * r - (_PBLK - 1))


def _lo_half(y):
    return lax.bitcast_convert_type(y << 16, jnp.float32)


def _hi_half(y):
    return lax.bitcast_convert_type(y & -65536, jnp.float32)


def _tc_body(w_ref, c_ref, o_ref):
    dn = (((1,), (1,)), ((), ()))
    y = c_ref[...]
    y_s = y[:, :FW]
    y_n = y[:, FW:]
    acc = lax.dot_general(w_ref[:, :FW], _lo_half(y_s), dn,
                          preferred_element_type=jnp.float32)
    acc += lax.dot_general(w_ref[:, FW:F], _hi_half(y_s), dn,
                           preferred_element_type=jnp.float32)
    acc += lax.dot_general(w_ref[:, F:F + FW], _lo_half(y_n), dn,
                           preferred_element_type=jnp.float32)
    acc += lax.dot_general(w_ref[:, F + FW:], _hi_half(y_n), dn,
                           preferred_element_type=jnp.float32)
    o_ref[...] = jnp.maximum(acc, 0.0)


_BLK = 8192


@jax.jit
def _tc_matmul(w, comb):
    return pl.pallas_call(
        _tc_body,
        out_shape=jax.ShapeDtypeStruct((E, B), jnp.float32),
        grid=(B // _BLK,),
        in_specs=[
            pl.BlockSpec((E, 2 * F), lambda i: (0, 0)),
            pl.BlockSpec((_BLK, F), lambda i: (i, 0)),
        ],
        out_specs=pl.BlockSpec((E, _BLK), lambda i: (0, i)),
    )(w, comb)


def kernel(nodes, neigh_idx, features, weight):
    nodes2 = _remap_idx(nodes.astype(jnp.int32)).reshape(B // 128, 128)
    neigh2 = _remap_idx(neigh_idx.astype(jnp.int32)).reshape(
        B * S // (CH * S), CH * S)
    fi = _tc_pack(features)
    comb = _build_sc_gather()(nodes2, neigh2, fi)
    wscaled = jnp.concatenate(
        [weight[:, :F], weight[:, F:] * (1.0 / S)], axis=1)
    return _tc_matmul(wscaled, comb)


# speedup vs baseline: 1.1961x; 1.1961x over previous
import functools

import jax
import jax.numpy as jnp
from jax import lax
from jax.experimental import pallas as pl
from jax.experimental.pallas import tpu as pltpu
from jax.experimental.pallas import tpu_sc as plsc

B = 16384
S = 25
F = 128
E = 128
L = 16
FW = F // 2
NC, NS = 2, 16
NW = NC * NS
BPW = B // NW
CH = 4
NCHUNK = BPW // CH
NBUF = 2
SG = B // (NW * 128)

def _lo_f32(x):
    return plsc.bitcast(x << 16, jnp.float32)


def _hi_f32(x):
    return plsc.bitcast(x & -65536, jnp.float32)


def _accum_chunk(nrows, obuf):
    for d in range(CH):
        r0 = d * S
        for j in range(FW // L):
            sl = pl.ds(j * L, L)
            acc = plsc.bitcast(nrows[r0, sl], jnp.bfloat16)
            for s in range(1, S):
                acc = acc + plsc.bitcast(nrows[r0 + s, sl], jnp.bfloat16)
            obuf[d, sl] = plsc.bitcast(acc, jnp.int32)


@functools.cache
def _build_sc_gather():
  mesh = plsc.VectorSubcoreMesh(core_axis_name="c", subcore_axis_name="s")

  @functools.partial(
    pl.kernel,
    out_type=jax.ShapeDtypeStruct((B, F), jnp.int32),
    mesh=mesh,
    compiler_params=pltpu.CompilerParams(
        needs_layout_passes=False, use_tc_tiling_on_sc=False),
    scratch_types=[
        pltpu.VMEM((SG, 128), jnp.int32),
        pltpu.VMEM((NCHUNK, CH * S), jnp.int32),
        pltpu.VMEM((2, 128, FW), jnp.int32),
        [pltpu.VMEM((CH * S, FW), jnp.int32)] * NBUF,
        [pltpu.VMEM((CH, FW), jnp.int32)] * NBUF,
        [pltpu.SemaphoreType.DMA] * NBUF,
        [pltpu.SemaphoreType.DMA] * NBUF,
        [pltpu.SemaphoreType.DMA] * 2,
        [pltpu.SemaphoreType.DMA] * 2,
    ],
)
  def _sc_gather(nodes2, neigh2, feat, comb_out,
                 nidx, eidx, srows, nrows, obufs, gsems, wsems, sgsems, swsems):
      wid = lax.axis_index("s") * NC + lax.axis_index("c")
      obase = wid * BPW

      pltpu.sync_copy(nodes2.at[pl.ds(wid * SG, SG)], nidx)
      pltpu.sync_copy(neigh2.at[pl.ds(wid * NCHUNK, NCHUNK)], eidx)

      for c in range(NBUF):
          pltpu.make_async_copy(feat.at[eidx.at[c]], nrows[c], gsems[c]).start()

      pltpu.make_async_copy(feat.at[nidx.at[0]], srows.at[0], sgsems[0]).start()
      pltpu.make_async_copy(feat.at[nidx.at[1]], srows.at[1], sgsems[1]).start()
      for g in range(SG):
          p = g % 2
          pltpu.make_async_copy(feat.at[nidx.at[g]], srows.at[p], sgsems[p]).wait()
          out_sl = comb_out.at[pl.ds(obase + g * 128, 128), pl.ds(0, FW)]
          pltpu.make_async_copy(srows.at[p], out_sl, swsems[p]).start()
          if g + 2 < SG:
              pltpu.make_async_copy(srows.at[p], out_sl, swsems[p]).wait()
              pltpu.make_async_copy(feat.at[nidx.at[g + 2]], srows.at[p], sgsems[p]).start()
      for g in range(SG - 2, SG):
          p = g % 2
          out_sl = comb_out.at[pl.ds(obase + g * 128, 128), pl.ds(0, FW)]
          pltpu.make_async_copy(srows.at[p], out_sl, swsems[p]).wait()

      def body(c2, carry):
          for k in range(NBUF):
              c = c2 * NBUF + k

              @pl.when(c >= NBUF)
              def _wait_write():
                  dst = comb_out.at[pl.ds(obase + (c - NBUF) * CH, CH),
                                    pl.ds(FW, FW)]
                  pltpu.make_async_copy(obufs[k], dst, wsems[k]).wait()

              pltpu.make_async_copy(feat.at[eidx.at[c]], nrows[k], gsems[k]).wait()
              _accum_chunk(nrows[k], obufs[k])

              @pl.when(c + NBUF < NCHUNK)
              def _next_gather():
                  pltpu.make_async_copy(
                      feat.at[eidx.at[c + NBUF]], nrows[k], gsems[k]).start()

              dst = comb_out.at[pl.ds(obase + c * CH, CH), pl.ds(FW, FW)]
              pltpu.make_async_copy(obufs[k], dst, wsems[k]).start()
          return carry

      lax.fori_loop(0, NCHUNK // NBUF, body, 0)

      for c in range(NCHUNK - NBUF, NCHUNK):
          k = c % NBUF
          dst = comb_out.at[pl.ds(obase + c * CH, CH), pl.ds(FW, FW)]
          pltpu.make_async_copy(obufs[k], dst, wsems[k]).wait()

  return _sc_gather


def _pack_body(x_ref, o_ref):
    u = lax.bitcast_convert_type(x_ref[...], jnp.int32)
    def rnd(v):
        lsb = lax.shift_right_logical(v, 16) & 1
        return lax.shift_right_logical(v + 0x7FFF + lsb, 16)
    w = rnd(u[:, :FW]) | (rnd(u[:, FW:]) << 16)
    o_ref[:, :FW] = w[:_PBLK // 2]
    o_ref[:, FW:] = w[_PBLK // 2:]


_PBLK = 10000
N_NODES_ = 100000


@jax.jit
def _tc_pack(features):
    packed = pl.pallas_call(
        _pack_body,
        out_shape=jax.ShapeDtypeStruct((N_NODES_ // 2, F), jnp.int32),
        grid=(N_NODES_ // _PBLK,),
        in_specs=[pl.BlockSpec((_PBLK, F), lambda i: (i, 0))],
        out_specs=pl.BlockSpec((_PBLK // 2, F), lambda i: (i, 0)),
    )(features)
    return packed.reshape(N_NODES_, FW)


def _remap_idx(n):
    r = n % _PBLK
    return (n - r) + jnp.where(r < _PBLK // 2, 2 * r, 2 * r - (_PBLK - 1))


def _lo_half(y):
    return lax.bitcast_convert_type(y << 16, jnp.float32)


def _hi_half(y):
    return lax.bitcast_convert_type(y & -65536, jnp.float32)


def _tc_body(w_ref, c_ref, o_ref):
    dn = (((1,), (1,)), ((), ()))
    y = c_ref[...]
    y_s = y[:, :FW]
    y_n = y[:, FW:]
    acc = lax.dot_general(w_ref[:, :FW], _lo_half(y_s), dn,
                          preferred_element_type=jnp.float32)
    acc += lax.dot_general(w_ref[:, FW:F], _hi_half(y_s), dn,
                           preferred_element_type=jnp.float32)
    acc += lax.dot_general(w_ref[:, F:F + FW], _lo_half(y_n), dn,
                           preferred_element_type=jnp.float32)
    acc += lax.dot_general(w_ref[:, F + FW:], _hi_half(y_n), dn,
                           preferred_element_type=jnp.float32)
    o_ref[...] = jnp.maximum(acc, 0.0)


_BLK = 8192


@jax.jit
def _tc_matmul(w, comb):
    return pl.pallas_call(
        _tc_body,
        out_shape=jax.ShapeDtypeStruct((E, B), jnp.float32),
        grid=(B // _BLK,),
        in_specs=[
            pl.BlockSpec((E, 2 * F), lambda i: (0, 0)),
            pl.BlockSpec((_BLK, F), lambda i: (i, 0)),
        ],
        out_specs=pl.BlockSpec((E, _BLK), lambda i: (0, i)),
    )(w, comb)


def kernel(nodes, neigh_idx, features, weight):
    nodes2 = _remap_idx(nodes.astype(jnp.int32)).reshape(B // 128, 128)
    neigh2 = _remap_idx(neigh_idx.astype(jnp.int32)).reshape(
        B * S // (CH * S), CH * S)
    fi = _tc_pack(features)
    comb = _build_sc_gather()(nodes2, neigh2, fi)
    wscaled = jnp.concatenate(
        [weight[:, :F], weight[:, F:] * (1.0 / S)], axis=1)
    return _tc_matmul(wscaled, comb)


# speedup vs baseline: 1.1980x; 1.0016x over previous
import functools

import jax
import jax.numpy as jnp
from jax import lax
from jax.experimental import pallas as pl
from jax.experimental.pallas import tpu as pltpu
from jax.experimental.pallas import tpu_sc as plsc

B = 16384
S = 25
F = 128
E = 128
L = 16
FW = F // 2
NC, NS = 2, 16
NW = NC * NS
BPW = B // NW
CH = 4
NCHUNK = BPW // CH
NBUF = 2
SG = B // (NW * 128)

def _lo_f32(x):
    return plsc.bitcast(x << 16, jnp.float32)


def _hi_f32(x):
    return plsc.bitcast(x & -65536, jnp.float32)


def _accum_chunk(nrows, obuf):
    for d in range(CH):
        r0 = d * S
        for j in range(FW // L):
            sl = pl.ds(j * L, L)
            acc = plsc.bitcast(nrows[r0, sl], jnp.bfloat16)
            for s in range(1, S):
                acc = acc + plsc.bitcast(nrows[r0 + s, sl], jnp.bfloat16)
            obuf[d, sl] = plsc.bitcast(acc, jnp.int32)


@functools.cache
def _build_sc_gather():
  mesh = plsc.VectorSubcoreMesh(core_axis_name="c", subcore_axis_name="s")

  @functools.partial(
    pl.kernel,
    out_type=jax.ShapeDtypeStruct((B, F), jnp.int32),
    mesh=mesh,
    compiler_params=pltpu.CompilerParams(
        needs_layout_passes=False, use_tc_tiling_on_sc=False),
    scratch_types=[
        pltpu.VMEM((SG, 128), jnp.int32),
        pltpu.VMEM((NCHUNK, CH * S), jnp.int32),
        pltpu.VMEM((2, 128, FW), jnp.int32),
        [pltpu.VMEM((CH * S, FW), jnp.int32)] * NBUF,
        [pltpu.VMEM((CH, FW), jnp.int32)] * NBUF,
        [pltpu.SemaphoreType.DMA] * NBUF,
        [pltpu.SemaphoreType.DMA] * NBUF,
        [pltpu.SemaphoreType.DMA] * 2,
        [pltpu.SemaphoreType.DMA] * 2,
    ],
)
  def _sc_gather(nodes2, neigh2, feat, comb_out,
                 nidx, eidx, srows, nrows, obufs, gsems, wsems, sgsems, swsems):
      wid = lax.axis_index("s") * NC + lax.axis_index("c")
      obase = wid * BPW

      pltpu.sync_copy(nodes2.at[pl.ds(wid * SG, SG)], nidx)
      pltpu.sync_copy(neigh2.at[pl.ds(wid * NCHUNK, NCHUNK)], eidx)

      for c in range(NBUF):
          pltpu.make_async_copy(feat.at[eidx.at[c]], nrows[c], gsems[c]).start()

      pltpu.make_async_copy(feat.at[nidx.at[0]], srows.at[0], sgsems[0]).start()
      pltpu.make_async_copy(feat.at[nidx.at[1]], srows.at[1], sgsems[1]).start()
      for g in range(SG):
          p = g % 2
          pltpu.make_async_copy(feat.at[nidx.at[g]], srows.at[p], sgsems[p]).wait()
          out_sl = comb_out.at[pl.ds(obase + g * 128, 128), pl.ds(0, FW)]
          pltpu.make_async_copy(srows.at[p], out_sl, swsems[p]).start()
          if g + 2 < SG:
              pltpu.make_async_copy(srows.at[p], out_sl, swsems[p]).wait()
              pltpu.make_async_copy(feat.at[nidx.at[g + 2]], srows.at[p], sgsems[p]).start()
      for g in range(SG - 2, SG):
          p = g % 2
          out_sl = comb_out.at[pl.ds(obase + g * 128, 128), pl.ds(0, FW)]
          pltpu.make_async_copy(srows.at[p], out_sl, swsems[p]).wait()

      def body(c2, carry):
          for k in range(NBUF):
              c = c2 * NBUF + k

              @pl.when(c >= NBUF)
              def _wait_write():
                  dst = comb_out.at[pl.ds(obase + (c - NBUF) * CH, CH),
                                    pl.ds(FW, FW)]
                  pltpu.make_async_copy(obufs[k], dst, wsems[k]).wait()

              pltpu.make_async_copy(feat.at[eidx.at[c]], nrows[k], gsems[k]).wait()
              _accum_chunk(nrows[k], obufs[k])

              @pl.when(c + NBUF < NCHUNK)
              def _next_gather():
                  pltpu.make_async_copy(
                      feat.at[eidx.at[c + NBUF]], nrows[k], gsems[k]).start()

              dst = comb_out.at[pl.ds(obase + c * CH, CH), pl.ds(FW, FW)]
              pltpu.make_async_copy(obufs[k], dst, wsems[k]).start()
          return carry

      lax.fori_loop(0, NCHUNK // NBUF, body, 0)

      for c in range(NCHUNK - NBUF, NCHUNK):
          k = c % NBUF
          dst = comb_out.at[pl.ds(obase + c * CH, CH), pl.ds(FW, FW)]
          pltpu.make_async_copy(obufs[k], dst, wsems[k]).wait()

  return _sc_gather


def _pack_body(x_ref, o_ref):
    u = lax.bitcast_convert_type(x_ref[...], jnp.int32)
    def rnd(v):
        lsb = lax.shift_right_logical(v, 16) & 1
        return lax.shift_right_logical(v + 0x7FFF + lsb, 16)
    w = rnd(u[:, :FW]) | (rnd(u[:, FW:]) << 16)
    o_ref[:, :FW] = w[:_PBLK // 2]
    o_ref[:, FW:] = w[_PBLK // 2:]


_PBLK = 20000
N_NODES_ = 100000


@jax.jit
def _tc_pack(features):
    packed = pl.pallas_call(
        _pack_body,
        out_shape=jax.ShapeDtypeStruct((N_NODES_ // 2, F), jnp.int32),
        grid=(N_NODES_ // _PBLK,),
        in_specs=[pl.BlockSpec((_PBLK, F), lambda i: (i, 0))],
        out_specs=pl.BlockSpec((_PBLK // 2, F), lambda i: (i, 0)),
    )(features)
    return packed.reshape(N_NODES_, FW)


def _remap_idx(n):
    r = n % _PBLK
    return (n - r) + jnp.where(r < _PBLK // 2, 2 * r, 2 * r - (_PBLK - 1))


def _lo_half(y):
    return lax.bitcast_convert_type(y << 16, jnp.float32)


def _hi_half(y):
    return lax.bitcast_convert_type(y & -65536, jnp.float32)


def _tc_body(w_ref, c_ref, o_ref):
    dn = (((1,), (1,)), ((), ()))
    y = c_ref[...]
    y_s = y[:, :FW]
    y_n = y[:, FW:]
    acc = lax.dot_general(w_ref[:, :FW], _lo_half(y_s), dn,
                          preferred_element_type=jnp.float32)
    acc += lax.dot_general(w_ref[:, FW:F], _hi_half(y_s), dn,
                           preferred_element_type=jnp.float32)
    acc += lax.dot_general(w_ref[:, F:F + FW], _lo_half(y_n), dn,
                           preferred_element_type=jnp.float32)
    acc += lax.dot_general(w_ref[:, F + FW:], _hi_half(y_n), dn,
                           preferred_element_type=jnp.float32)
    o_ref[...] = jnp.maximum(acc, 0.0)


_BLK = 8192


@jax.jit
def _tc_matmul(w, comb):
    return pl.pallas_call(
        _tc_body,
        out_shape=jax.ShapeDtypeStruct((E, B), jnp.float32),
        grid=(B // _BLK,),
        in_specs=[
            pl.BlockSpec((E, 2 * F), lambda i: (0, 0)),
            pl.BlockSpec((_BLK, F), lambda i: (i, 0)),
        ],
        out_specs=pl.BlockSpec((E, _BLK), lambda i: (0, i)),
    )(w, comb)


def kernel(nodes, neigh_idx, features, weight):
    nodes2 = _remap_idx(nodes.astype(jnp.int32)).reshape(B // 128, 128)
    neigh2 = _remap_idx(neigh_idx.astype(jnp.int32)).reshape(
        B * S // (CH * S), CH * S)
    fi = _tc_pack(features)
    comb = _build_sc_gather()(nodes2, neigh2, fi)
    wscaled = jnp.concatenate(
        [weight[:, :F], weight[:, F:] * (1.0 / S)], axis=1)
    return _tc_matmul(wscaled, comb)
